# baseline (speedup 1.0000x reference)
import functools

import jax
import jax.numpy as jnp
from jax import lax
from jax.experimental import pallas as pl
from jax.experimental.pallas import tpu as pltpu
from jax.experimental.pallas import tpu_sc as plsc

NC = 2
NS = 16
NW = NC * NS
CHUNK = 128
NBUF = 3

_mesh = plsc.VectorSubcoreMesh(core_axis_name="c", subcore_axis_name="s")


def _round_up(a, m):
    return ((a + m - 1) // m) * m


def _fill(buf, rows, d, value):
    @pl.loop(0, rows)
    def _(i):
        @pl.loop(0, d // 16)
        def _(j):
            buf[pl.ds(i, 1), pl.ds(j * 16, 16)] = jnp.full((1, 16), value,
                                                           jnp.float32)


def _zero_stripe(zbuf, acc, sid, stripe, d):
    full, rem = stripe // CHUNK, stripe % CHUNK

    @pl.loop(0, full)
    def _(t):
        pltpu.sync_copy(zbuf, acc.at[pl.ds(sid * stripe + t * CHUNK, CHUNK)])

    if rem:
        pltpu.sync_copy(zbuf.at[pl.ds(0, rem)],
                        acc.at[pl.ds(sid * stripe + full * CHUNK, rem)])


def _read_stripe(acc, out_hbm, cid, sid, stripe):
    full, rem = stripe // CHUNK, stripe % CHUNK

    @pl.loop(0, full)
    def _(t):
        r0 = sid * stripe + t * CHUNK
        pltpu.sync_copy(acc.at[pl.ds(r0, CHUNK)], out_hbm.at[cid, pl.ds(r0, CHUNK)])

    if rem:
        r0 = sid * stripe + full * CHUNK
        pltpu.sync_copy(acc.at[pl.ds(r0, rem)], out_hbm.at[cid, pl.ds(r0, rem)])


def _make_count(npad, epad):
    per_w = epad // NW
    n_chunks = per_w // CHUNK
    n_rounds = n_chunks // NBUF
    stripe = npad // NS
    d = 128

    @functools.partial(
        pl.kernel,
        out_type=jax.ShapeDtypeStruct((NC, npad, d), jnp.float32),
        mesh=_mesh,
        scratch_types=[pltpu.VMEM((CHUNK, d), jnp.float32),
                       pltpu.VMEM_SHARED((npad, d), jnp.float32)]
        + [pltpu.VMEM((CHUNK,), jnp.int32) for _ in range(NBUF)]
        + [pltpu.SemaphoreType.DMA for _ in range(2 * NBUF)],
    )
    def count(row_hbm, out_hbm, ones_v, acc, *rest):
        ridx = rest[:NBUF]
        risem = rest[NBUF:2 * NBUF]
        ssem = rest[2 * NBUF:]

        cid = lax.axis_index("c")
        sid = lax.axis_index("s")
        wid = sid * NC + cid
        base = wid * per_w

        for b in range(NBUF):
            pltpu.async_copy(row_hbm.at[pl.ds(base + b * CHUNK, CHUNK)],
                             ridx[b], risem[b])

        _fill(ones_v, CHUNK, d, 0.0)
        _zero_stripe(ones_v, acc, sid, stripe, d)
        _fill(ones_v, CHUNK, d, 1.0 / d)
        plsc.subcore_barrier()

        @pl.loop(0, n_rounds)
        def _(r):
            c0 = r * NBUF
            for b in range(NBUF):
                c = c0 + b
                pltpu.make_async_copy(
                    row_hbm.at[pl.ds(base + c * CHUNK, CHUNK)], ridx[b],
                    risem[b]).wait()
                pltpu.async_copy(ones_v, acc.at[ridx[b]], ssem[b], add=True)
            for b in range(NBUF):
                c = c0 + b

                @pl.when(r < n_rounds - 1)
                def _():
                    pltpu.make_async_copy(ones_v, acc.at[ridx[b]],
                                          ssem[b]).wait()
                    pltpu.async_copy(
                        row_hbm.at[pl.ds(base + (c + NBUF) * CHUNK, CHUNK)],
                        ridx[b], risem[b])

        for b in range(NBUF):
            pltpu.make_async_copy(ones_v, acc.at[ridx[b]], ssem[b]).wait()

        plsc.subcore_barrier()
        _read_stripe(acc, out_hbm, cid, sid, stripe)

    return count


def _make_scatter(npad, epad, d):
    per_w = epad // NW
    n_chunks = per_w // CHUNK
    n_rounds = n_chunks // NBUF
    stripe = npad // NS

    @functools.partial(
        pl.kernel,
        out_type=jax.ShapeDtypeStruct((NC, npad, d), jnp.float32),
        mesh=_mesh,
        scratch_types=[pltpu.VMEM_SHARED((npad, d), jnp.float32)]
        + [pltpu.VMEM((CHUNK, d), jnp.float32) for _ in range(NBUF)]
        + [pltpu.VMEM((CHUNK,), jnp.int32) for _ in range(2 * NBUF)]
        + [pltpu.SemaphoreType.DMA for _ in range(4 * NBUF)],
    )
    def scatter(h_hbm, row_hbm, col_hbm, out_hbm, acc, *rest):
        rows = rest[:NBUF]
        ridx = rest[NBUF:2 * NBUF]
        cidx = rest[2 * NBUF:3 * NBUF]
        risem = rest[3 * NBUF:4 * NBUF]
        cisem = rest[4 * NBUF:5 * NBUF]
        gsem = rest[5 * NBUF:6 * NBUF]
        ssem = rest[6 * NBUF:]

        cid = lax.axis_index("c")
        sid = lax.axis_index("s")
        wid = sid * NC + cid
        base = wid * per_w

        def g_cp(b):
            return pltpu.make_async_copy(h_hbm.at[ridx[b]], rows[b], gsem[b])

        def s_cp(b):
            return pltpu.make_async_copy(rows[b], acc.at[cidx[b]], ssem[b])

        for b in range(NBUF):
            pltpu.async_copy(row_hbm.at[pl.ds(base + b * CHUNK, CHUNK)],
                             ridx[b], risem[b])
            pltpu.async_copy(col_hbm.at[pl.ds(base + b * CHUNK, CHUNK)],
                             cidx[b], cisem[b])

        _fill(rows[0], CHUNK, d, 0.0)
        _zero_stripe(rows[0], acc, sid, stripe, d)

        for b in range(NBUF):
            pltpu.make_async_copy(
                row_hbm.at[pl.ds(base + b * CHUNK, CHUNK)], ridx[b],
                risem[b]).wait()
            pltpu.async_copy(h_hbm.at[ridx[b]], rows[b], gsem[b])

        plsc.subcore_barrier()

        @pl.loop(0, n_rounds)
        def _(r):
            c0 = r * NBUF
            for b in range(NBUF):
                c = c0 + b
                g_cp(b).wait()
                pltpu.make_async_copy(
                    col_hbm.at[pl.ds(base + c * CHUNK, CHUNK)], cidx[b],
                    cisem[b]).wait()
                pltpu.async_copy(rows[b], acc.at[cidx[b]], ssem[b], add=True)

                @pl.when(r < n_rounds - 1)
                def _():
                    pltpu.async_copy(
                        row_hbm.at[pl.ds(base + (c + NBUF) * CHUNK, CHUNK)],
                        ridx[b], risem[b])

            for b in range(NBUF):
                c = c0 + b

                @pl.when(r < n_rounds - 1)
                def _():
                    s_cp(b).wait()
                    pltpu.async_copy(
                        col_hbm.at[pl.ds(base + (c + NBUF) * CHUNK, CHUNK)],
                        cidx[b], cisem[b])
                    pltpu.make_async_copy(
                        row_hbm.at[pl.ds(base + (c + NBUF) * CHUNK, CHUNK)],
                        ridx[b], risem[b]).wait()
                    pltpu.async_copy(h_hbm.at[ridx[b]], rows[b], gsem[b])

        for b in range(NBUF):
            s_cp(b).wait()

        plsc.subcore_barrier()
        _read_stripe(acc, out_hbm, cid, sid, stripe)

    return scatter


def _tc_matmul(x, w):
    def body(x_ref, w_ref, o_ref):
        o_ref[...] = jnp.dot(x_ref[...], w_ref[...],
                             preferred_element_type=jnp.float32)

    return pl.pallas_call(
        body,
        out_shape=jax.ShapeDtypeStruct((x.shape[0], w.shape[1]), jnp.float32),
    )(x, w)


def _dis(degp_ref):
    deg = jnp.sum(degp_ref[0] + degp_ref[1], axis=1, keepdims=True) + 1.0
    return lax.rsqrt(deg)


def _tc_scale(xw, degp):
    def body(xw_ref, degp_ref, o_ref):
        o_ref[...] = _dis(degp_ref) * xw_ref[...]

    return pl.pallas_call(
        body,
        out_shape=jax.ShapeDtypeStruct(xw.shape, jnp.float32),
    )(xw, degp)


def _tc_layer(sp, hp, degp, b, w):
    def body(sp_ref, hp_ref, degp_ref, b_ref, w_ref, o_ref):
        dis = _dis(degp_ref)
        aggr = dis * (sp_ref[0] + sp_ref[1] + hp_ref[...]) + b_ref[...]
        h = jnp.maximum(aggr, 0.0)
        o_ref[...] = dis * jnp.dot(h, w_ref[...],
                                   preferred_element_type=jnp.float32)

    return pl.pallas_call(
        body,
        out_shape=jax.ShapeDtypeStruct((hp.shape[0], w.shape[1]), jnp.float32),
    )(sp, hp, degp, b, w)


def _tc_finish(sp, hp, degp, b):
    def body(sp_ref, hp_ref, degp_ref, b_ref, o_ref):
        dis = _dis(degp_ref)
        o_ref[...] = dis * (sp_ref[0] + sp_ref[1] + hp_ref[...]) + b_ref[...]

    return pl.pallas_call(
        body,
        out_shape=jax.ShapeDtypeStruct(hp.shape, jnp.float32),
    )(sp, hp, degp, b)


def kernel(x, edge_index, W1, b1, W2, b2):
    n, in_ch = x.shape
    e = edge_index.shape[1]
    npad = _round_up(n + 1, CHUNK)
    epad = _round_up(e, NW * CHUNK * NBUF)

    out_ch = W2.shape[1]
    oc_pad = _round_up(out_ch, 128)

    xpad = jnp.zeros((npad, in_ch), x.dtype).at[:n].set(x)
    pad_idx = jnp.full((epad - e,), n, jnp.int32)
    rowp = jnp.concatenate([edge_index[0], pad_idx])
    colp = jnp.concatenate([edge_index[1], pad_idx])
    b1r = b1.reshape(1, -1)
    b2r = jnp.zeros((1, oc_pad), b2.dtype).at[0, :out_ch].set(b2)
    W2p = jnp.zeros((W2.shape[0], oc_pad), W2.dtype).at[:, :out_ch].set(W2)

    count = _make_count(npad, epad)
    scat1 = _make_scatter(npad, epad, W1.shape[1])
    scat2 = _make_scatter(npad, epad, oc_pad)

    degp = count(rowp)
    xw = _tc_matmul(xpad, W1)
    hp1 = _tc_scale(xw, degp)
    s1 = scat1(hp1, rowp, colp)
    hp2 = _tc_layer(s1, hp1, degp, b1r, W2p)
    s2 = scat2(hp2, rowp, colp)
    outp = _tc_finish(s2, hp2, degp, b2r)
    return outp[:n, :out_ch]


# speedup vs baseline: 1.2965x; 1.2965x over previous
import functools

import jax
import jax.numpy as jnp
from jax import lax
from jax.experimental import pallas as pl
from jax.experimental.pallas import tpu as pltpu
from jax.experimental.pallas import tpu_sc as plsc

NC = 2
NS = 16
NW = NC * NS
CHUNK = 128
NBUF = 3

_mesh = plsc.VectorSubcoreMesh(core_axis_name="c", subcore_axis_name="s")


def _round_up(a, m):
    return ((a + m - 1) // m) * m


def _fill(buf, rows, d, value):
    @pl.loop(0, rows)
    def _(i):
        @pl.loop(0, d // 16)
        def _(j):
            buf[pl.ds(i, 1), pl.ds(j * 16, 16)] = jnp.full((1, 16), value,
                                                           jnp.float32)


def _zero_stripe(zbuf, acc, sid, stripe, d, chunk=CHUNK):
    full, rem = stripe // chunk, stripe % chunk

    @pl.loop(0, full)
    def _(t):
        pltpu.sync_copy(zbuf, acc.at[pl.ds(sid * stripe + t * chunk, chunk)])

    if rem:
        pltpu.sync_copy(zbuf.at[pl.ds(0, rem)],
                        acc.at[pl.ds(sid * stripe + full * chunk, rem)])


def _read_stripe(acc, out_hbm, cid, sid, stripe):
    full, rem = stripe // CHUNK, stripe % CHUNK

    @pl.loop(0, full)
    def _(t):
        r0 = sid * stripe + t * CHUNK
        pltpu.sync_copy(acc.at[pl.ds(r0, CHUNK)], out_hbm.at[cid, pl.ds(r0, CHUNK)])

    if rem:
        r0 = sid * stripe + full * CHUNK
        pltpu.sync_copy(acc.at[pl.ds(r0, rem)], out_hbm.at[cid, pl.ds(r0, rem)])


def _make_count(npad, epad):
    per_w = epad // NW
    n_chunks = per_w // CHUNK
    n_rounds = n_chunks // NBUF
    stripe = npad // NS
    d = 128

    @functools.partial(
        pl.kernel,
        out_type=jax.ShapeDtypeStruct((NC, npad, d), jnp.float32),
        mesh=_mesh,
        scratch_types=[pltpu.VMEM((CHUNK, d), jnp.float32),
                       pltpu.VMEM_SHARED((npad, d), jnp.float32)]
        + [pltpu.VMEM((CHUNK,), jnp.int32) for _ in range(NBUF)]
        + [pltpu.SemaphoreType.DMA for _ in range(2 * NBUF)],
    )
    def count(row_hbm, out_hbm, ones_v, acc, *rest):
        ridx = rest[:NBUF]
        risem = rest[NBUF:2 * NBUF]
        ssem = rest[2 * NBUF:]

        cid = lax.axis_index("c")
        sid = lax.axis_index("s")
        wid = sid * NC + cid
        base = wid * per_w

        for b in range(NBUF):
            pltpu.async_copy(row_hbm.at[pl.ds(base + b * CHUNK, CHUNK)],
                             ridx[b], risem[b])

        _fill(ones_v, CHUNK, d, 0.0)
        _zero_stripe(ones_v, acc, sid, stripe, d)
        _fill(ones_v, CHUNK, d, 1.0 / d)
        plsc.subcore_barrier()

        @pl.loop(0, n_rounds)
        def _(r):
            c0 = r * NBUF
            for b in range(NBUF):
                c = c0 + b
                pltpu.make_async_copy(
                    row_hbm.at[pl.ds(base + c * CHUNK, CHUNK)], ridx[b],
                    risem[b]).wait()
                pltpu.async_copy(ones_v, acc.at[ridx[b]], ssem[b], add=True)
            for b in range(NBUF):
                c = c0 + b

                @pl.when(r < n_rounds - 1)
                def _():
                    pltpu.make_async_copy(ones_v, acc.at[ridx[b]],
                                          ssem[b]).wait()
                    pltpu.async_copy(
                        row_hbm.at[pl.ds(base + (c + NBUF) * CHUNK, CHUNK)],
                        ridx[b], risem[b])

        for b in range(NBUF):
            pltpu.make_async_copy(ones_v, acc.at[ridx[b]], ssem[b]).wait()

        plsc.subcore_barrier()
        _read_stripe(acc, out_hbm, cid, sid, stripe)

    return count


def _make_scatter(npad, epad, d, chunk=64, nbuf=5):
    per_w = epad // NW
    n_chunks = per_w // chunk
    n_rounds = n_chunks // nbuf
    stripe = npad // NS
    CHUNK, NBUF = chunk, nbuf

    @functools.partial(
        pl.kernel,
        out_type=jax.ShapeDtypeStruct((NC, npad, d), jnp.float32),
        mesh=_mesh,
        scratch_types=[pltpu.VMEM_SHARED((npad, d), jnp.float32)]
        + [pltpu.VMEM((CHUNK, d), jnp.float32) for _ in range(NBUF)]
        + [pltpu.VMEM((CHUNK,), jnp.int32) for _ in range(2 * NBUF)]
        + [pltpu.SemaphoreType.DMA for _ in range(4 * NBUF)],
    )
    def scatter(h_hbm, row_hbm, col_hbm, out_hbm, acc, *rest):
        rows = rest[:NBUF]
        ridx = rest[NBUF:2 * NBUF]
        cidx = rest[2 * NBUF:3 * NBUF]
        risem = rest[3 * NBUF:4 * NBUF]
        cisem = rest[4 * NBUF:5 * NBUF]
        gsem = rest[5 * NBUF:6 * NBUF]
        ssem = rest[6 * NBUF:]

        cid = lax.axis_index("c")
        sid = lax.axis_index("s")
        wid = sid * NC + cid
        base = wid * per_w

        def g_cp(b):
            return pltpu.make_async_copy(h_hbm.at[ridx[b]], rows[b], gsem[b])

        def s_cp(b):
            return pltpu.make_async_copy(rows[b], acc.at[cidx[b]], ssem[b])

        for b in range(NBUF):
            pltpu.async_copy(row_hbm.at[pl.ds(base + b * CHUNK, CHUNK)],
                             ridx[b], risem[b])
            pltpu.async_copy(col_hbm.at[pl.ds(base + b * CHUNK, CHUNK)],
                             cidx[b], cisem[b])

        _fill(rows[0], CHUNK, d, 0.0)
        _zero_stripe(rows[0], acc, sid, stripe, d, chunk=CHUNK)

        for b in range(NBUF):
            pltpu.make_async_copy(
                row_hbm.at[pl.ds(base + b * CHUNK, CHUNK)], ridx[b],
                risem[b]).wait()
            pltpu.async_copy(h_hbm.at[ridx[b]], rows[b], gsem[b])

        plsc.subcore_barrier()

        @pl.loop(0, n_rounds)
        def _(r):
            c0 = r * NBUF
            for b in range(NBUF):
                c = c0 + b
                g_cp(b).wait()
                pltpu.make_async_copy(
                    col_hbm.at[pl.ds(base + c * CHUNK, CHUNK)], cidx[b],
                    cisem[b]).wait()
                pltpu.async_copy(rows[b], acc.at[cidx[b]], ssem[b], add=True)

                @pl.when(r < n_rounds - 1)
                def _():
                    pltpu.async_copy(
                        row_hbm.at[pl.ds(base + (c + NBUF) * CHUNK, CHUNK)],
                        ridx[b], risem[b])

            for b in range(NBUF):
                c = c0 + b

                @pl.when(r < n_rounds - 1)
                def _():
                    s_cp(b).wait()
                    pltpu.async_copy(
                        col_hbm.at[pl.ds(base + (c + NBUF) * CHUNK, CHUNK)],
                        cidx[b], cisem[b])
                    pltpu.make_async_copy(
                        row_hbm.at[pl.ds(base + (c + NBUF) * CHUNK, CHUNK)],
                        ridx[b], risem[b]).wait()
                    pltpu.async_copy(h_hbm.at[ridx[b]], rows[b], gsem[b])

        for b in range(NBUF):
            s_cp(b).wait()

        plsc.subcore_barrier()
        _read_stripe(acc, out_hbm, cid, sid, stripe)

    return scatter


def _tc_matmul(x, w):
    def body(x_ref, w_ref, o_ref):
        o_ref[...] = jnp.dot(x_ref[...], w_ref[...],
                             preferred_element_type=jnp.float32)

    return pl.pallas_call(
        body,
        out_shape=jax.ShapeDtypeStruct((x.shape[0], w.shape[1]), jnp.float32),
    )(x, w)


def _dis(degp_ref):
    deg = jnp.sum(degp_ref[0] + degp_ref[1], axis=1, keepdims=True) + 1.0
    return lax.rsqrt(deg)


def _tc_scale(xw, degp):
    def body(xw_ref, degp_ref, o_ref):
        o_ref[...] = _dis(degp_ref) * xw_ref[...]

    return pl.pallas_call(
        body,
        out_shape=jax.ShapeDtypeStruct(xw.shape, jnp.float32),
    )(xw, degp)


def _tc_layer(sp, hp, degp, b, w):
    def body(sp_ref, hp_ref, degp_ref, b_ref, w_ref, o_ref):
        dis = _dis(degp_ref)
        aggr = dis * (sp_ref[0] + sp_ref[1] + hp_ref[...]) + b_ref[...]
        h = jnp.maximum(aggr, 0.0)
        o_ref[...] = dis * jnp.dot(h, w_ref[...],
                                   preferred_element_type=jnp.float32)

    return pl.pallas_call(
        body,
        out_shape=jax.ShapeDtypeStruct((hp.shape[0], w.shape[1]), jnp.float32),
    )(sp, hp, degp, b, w)


def _tc_finish(sp, hp, degp, b):
    def body(sp_ref, hp_ref, degp_ref, b_ref, o_ref):
        dis = _dis(degp_ref)
        o_ref[...] = dis * (sp_ref[0] + sp_ref[1] + hp_ref[...]) + b_ref[...]

    return pl.pallas_call(
        body,
        out_shape=jax.ShapeDtypeStruct(hp.shape, jnp.float32),
    )(sp, hp, degp, b)


def kernel(x, edge_index, W1, b1, W2, b2):
    n, in_ch = x.shape
    e = edge_index.shape[1]
    npad = _round_up(n + 1, CHUNK)
    epad_c = _round_up(e, NW * CHUNK * NBUF)
    epad_s = _round_up(e, NW * 64 * 5)
    epad = max(epad_c, epad_s)

    out_ch = W2.shape[1]
    oc_pad = _round_up(out_ch, 128)

    xpad = jnp.zeros((npad, in_ch), x.dtype).at[:n].set(x)
    pad_idx = jnp.full((epad - e,), n, jnp.int32)
    rowp = jnp.concatenate([edge_index[0], pad_idx])
    colp = jnp.concatenate([edge_index[1], pad_idx])
    b1r = b1.reshape(1, -1)
    b2r = jnp.zeros((1, oc_pad), b2.dtype).at[0, :out_ch].set(b2)
    W2p = jnp.zeros((W2.shape[0], oc_pad), W2.dtype).at[:, :out_ch].set(W2)

    count = _make_count(npad, epad_c)
    scat1 = _make_scatter(npad, epad_s, W1.shape[1])
    scat2 = _make_scatter(npad, epad_s, oc_pad)

    degp = count(rowp)
    xw = _tc_matmul(xpad, W1)
    hp1 = _tc_scale(xw, degp)
    s1 = scat1(hp1, rowp, colp)
    hp2 = _tc_layer(s1, hp1, degp, b1r, W2p)
    s2 = scat2(hp2, rowp, colp)
    outp = _tc_finish(s2, hp2, degp, b2r)
    return outp[:n, :out_ch]


# speedup vs baseline: 2.4213x; 1.8676x over previous
import functools

import jax
import jax.numpy as jnp
from jax import lax
from jax.experimental import pallas as pl
from jax.experimental.pallas import tpu as pltpu
from jax.experimental.pallas import tpu_sc as plsc

NC = 2
NS = 16
NW = NC * NS
CHUNK = 128
NBUF = 3

_mesh = plsc.VectorSubcoreMesh(core_axis_name="c", subcore_axis_name="s")


def _round_up(a, m):
    return ((a + m - 1) // m) * m


def _fill(buf, rows, d, value):
    @pl.loop(0, rows)
    def _(i):
        @pl.loop(0, d // 16)
        def _(j):
            buf[pl.ds(i, 1), pl.ds(j * 16, 16)] = jnp.full((1, 16), value,
                                                           jnp.float32)


def _zero_stripe(zbuf, acc, sid, stripe, d, chunk=CHUNK):
    full, rem = stripe // chunk, stripe % chunk

    @pl.loop(0, full)
    def _(t):
        pltpu.sync_copy(zbuf, acc.at[pl.ds(sid * stripe + t * chunk, chunk)])

    if rem:
        pltpu.sync_copy(zbuf.at[pl.ds(0, rem)],
                        acc.at[pl.ds(sid * stripe + full * chunk, rem)])


def _read_stripe(acc, out_hbm, cid, sid, stripe):
    full, rem = stripe // CHUNK, stripe % CHUNK

    @pl.loop(0, full)
    def _(t):
        r0 = sid * stripe + t * CHUNK
        pltpu.sync_copy(acc.at[pl.ds(r0, CHUNK)], out_hbm.at[cid, pl.ds(r0, CHUNK)])

    if rem:
        r0 = sid * stripe + full * CHUNK
        pltpu.sync_copy(acc.at[pl.ds(r0, rem)], out_hbm.at[cid, pl.ds(r0, rem)])


def _make_count(npad, epad):
    per_w = epad // NW
    n_chunks = per_w // CHUNK
    n_rounds = n_chunks // NBUF
    stripe = npad // NS
    d = 128

    @functools.partial(
        pl.kernel,
        out_type=jax.ShapeDtypeStruct((NC, npad, d), jnp.float32),
        mesh=_mesh,
        scratch_types=[pltpu.VMEM((CHUNK, d), jnp.float32),
                       pltpu.VMEM_SHARED((npad, d), jnp.float32)]
        + [pltpu.VMEM((CHUNK,), jnp.int32) for _ in range(NBUF)]
        + [pltpu.SemaphoreType.DMA for _ in range(2 * NBUF)],
    )
    def count(row_hbm, out_hbm, ones_v, acc, *rest):
        ridx = rest[:NBUF]
        risem = rest[NBUF:2 * NBUF]
        ssem = rest[2 * NBUF:]

        cid = lax.axis_index("c")
        sid = lax.axis_index("s")
        wid = sid * NC + cid
        base = wid * per_w

        for b in range(NBUF):
            pltpu.async_copy(row_hbm.at[pl.ds(base + b * CHUNK, CHUNK)],
                             ridx[b], risem[b])

        _fill(ones_v, CHUNK, d, 0.0)
        _zero_stripe(ones_v, acc, sid, stripe, d)
        _fill(ones_v, CHUNK, d, 1.0 / d)
        plsc.subcore_barrier()

        @pl.loop(0, n_rounds)
        def _(r):
            c0 = r * NBUF
            for b in range(NBUF):
                c = c0 + b
                pltpu.make_async_copy(
                    row_hbm.at[pl.ds(base + c * CHUNK, CHUNK)], ridx[b],
                    risem[b]).wait()
                pltpu.async_copy(ones_v, acc.at[ridx[b]], ssem[b], add=True)
            for b in range(NBUF):
                c = c0 + b

                @pl.when(r < n_rounds - 1)
                def _():
                    pltpu.make_async_copy(ones_v, acc.at[ridx[b]],
                                          ssem[b]).wait()
                    pltpu.async_copy(
                        row_hbm.at[pl.ds(base + (c + NBUF) * CHUNK, CHUNK)],
                        ridx[b], risem[b])

        for b in range(NBUF):
            pltpu.make_async_copy(ones_v, acc.at[ridx[b]], ssem[b]).wait()

        plsc.subcore_barrier()
        _read_stripe(acc, out_hbm, cid, sid, stripe)

    return count


def _make_scatter(npad, epad, d, chunk=48, nbuf=7):
    per_w = epad // NW
    n_chunks = per_w // chunk
    n_rounds = n_chunks // nbuf
    stripe = npad // NS
    CHUNK, NBUF = chunk, nbuf

    @functools.partial(
        pl.kernel,
        out_type=jax.ShapeDtypeStruct((NC, npad, d), jnp.float32),
        mesh=_mesh,
        scratch_types=[pltpu.VMEM_SHARED((npad, d), jnp.float32)]
        + [pltpu.VMEM((CHUNK, d), jnp.float32) for _ in range(NBUF)]
        + [pltpu.VMEM((CHUNK,), jnp.int32) for _ in range(2 * NBUF)]
        + [pltpu.SemaphoreType.DMA for _ in range(4 * NBUF)],
    )
    def scatter(h_hbm, row_hbm, col_hbm, out_hbm, acc, *rest):
        rows = rest[:NBUF]
        ridx = rest[NBUF:2 * NBUF]
        cidx = rest[2 * NBUF:3 * NBUF]
        risem = rest[3 * NBUF:4 * NBUF]
        cisem = rest[4 * NBUF:5 * NBUF]
        gsem = rest[5 * NBUF:6 * NBUF]
        ssem = rest[6 * NBUF:]

        cid = lax.axis_index("c")
        sid = lax.axis_index("s")
        wid = sid * NC + cid
        base = wid * per_w

        def g_cp(b):
            return pltpu.make_async_copy(h_hbm.at[ridx[b]], rows[b], gsem[b])

        def s_cp(b):
            return pltpu.make_async_copy(rows[b], acc.at[cidx[b]], ssem[b])

        for b in range(NBUF):
            pltpu.async_copy(row_hbm.at[pl.ds(base + b * CHUNK, CHUNK)],
                             ridx[b], risem[b])
            pltpu.async_copy(col_hbm.at[pl.ds(base + b * CHUNK, CHUNK)],
                             cidx[b], cisem[b])

        _fill(rows[0], CHUNK, d, 0.0)
        _zero_stripe(rows[0], acc, sid, stripe, d, chunk=CHUNK)

        for b in range(NBUF):
            pltpu.make_async_copy(
                row_hbm.at[pl.ds(base + b * CHUNK, CHUNK)], ridx[b],
                risem[b]).wait()
            pltpu.async_copy(h_hbm.at[ridx[b]], rows[b], gsem[b])

        plsc.subcore_barrier()

        @pl.loop(0, n_rounds)
        def _(r):
            c0 = r * NBUF
            for b in range(NBUF):
                c = c0 + b
                g_cp(b).wait()
                pltpu.make_async_copy(
                    col_hbm.at[pl.ds(base + c * CHUNK, CHUNK)], cidx[b],
                    cisem[b]).wait()
                pltpu.async_copy(rows[b], acc.at[cidx[b]], ssem[b], add=True)

                @pl.when(r < n_rounds - 1)
                def _():
                    pltpu.async_copy(
                        row_hbm.at[pl.ds(base + (c + NBUF) * CHUNK, CHUNK)],
                        ridx[b], risem[b])

            for b in range(NBUF):
                c = c0 + b

                @pl.when(r < n_rounds - 1)
                def _():
                    s_cp(b).wait()
                    pltpu.async_copy(
                        col_hbm.at[pl.ds(base + (c + NBUF) * CHUNK, CHUNK)],
                        cidx[b], cisem[b])
                    pltpu.make_async_copy(
                        row_hbm.at[pl.ds(base + (c + NBUF) * CHUNK, CHUNK)],
                        ridx[b], risem[b]).wait()
                    pltpu.async_copy(h_hbm.at[ridx[b]], rows[b], gsem[b])

        for b in range(NBUF):
            s_cp(b).wait()

        plsc.subcore_barrier()
        _read_stripe(acc, out_hbm, cid, sid, stripe)

    return scatter


def _tc_matmul(x, w):
    def body(x_ref, w_ref, o_ref):
        o_ref[...] = jnp.dot(x_ref[...], w_ref[...],
                             preferred_element_type=jnp.float32)

    return pl.pallas_call(
        body,
        out_shape=jax.ShapeDtypeStruct((x.shape[0], w.shape[1]), jnp.float32),
    )(x, w)


def _dis(degp_ref):
    deg = jnp.sum(degp_ref[0] + degp_ref[1], axis=1, keepdims=True) + 1.0
    return lax.rsqrt(deg)


def _tc_scale(xw, degp):
    def body(xw_ref, degp_ref, o_ref):
        o_ref[...] = _dis(degp_ref) * xw_ref[...]

    return pl.pallas_call(
        body,
        out_shape=jax.ShapeDtypeStruct(xw.shape, jnp.float32),
    )(xw, degp)


def _tc_layer(sp, hp, degp, b, w):
    def body(sp_ref, hp_ref, degp_ref, b_ref, w_ref, o_ref):
        dis = _dis(degp_ref)
        aggr = dis * (sp_ref[0] + sp_ref[1] + hp_ref[...]) + b_ref[...]
        h = jnp.maximum(aggr, 0.0)
        o_ref[...] = dis * jnp.dot(h, w_ref[...],
                                   preferred_element_type=jnp.float32)

    return pl.pallas_call(
        body,
        out_shape=jax.ShapeDtypeStruct((hp.shape[0], w.shape[1]), jnp.float32),
    )(sp, hp, degp, b, w)


def _tc_finish(sp, hp, degp, b):
    def body(sp_ref, hp_ref, degp_ref, b_ref, o_ref):
        dis = _dis(degp_ref)
        o_ref[...] = dis * (sp_ref[0] + sp_ref[1] + hp_ref[...]) + b_ref[...]

    return pl.pallas_call(
        body,
        out_shape=jax.ShapeDtypeStruct(hp.shape, jnp.float32),
    )(sp, hp, degp, b)


def kernel(x, edge_index, W1, b1, W2, b2):
    n, in_ch = x.shape
    e = edge_index.shape[1]
    npad = _round_up(n + 1, CHUNK)
    epad_c = _round_up(e, NW * CHUNK * NBUF)
    epad_s = _round_up(e, NW * 48 * 7)
    epad = max(epad_c, epad_s)

    out_ch = W2.shape[1]
    oc_pad = _round_up(out_ch, 128)

    xpad = jnp.zeros((npad, in_ch), x.dtype).at[:n].set(x)
    pad_idx = jnp.full((epad - e,), n, jnp.int32)
    rowp = jnp.concatenate([edge_index[0], pad_idx])
    colp = jnp.concatenate([edge_index[1], pad_idx])
    b1r = b1.reshape(1, -1)
    b2r = jnp.zeros((1, oc_pad), b2.dtype).at[0, :out_ch].set(b2)
    W2p = jnp.zeros((W2.shape[0], oc_pad), W2.dtype).at[:, :out_ch].set(W2)

    count = _make_count(npad, epad_c)
    scat1 = _make_scatter(npad, epad_s, W1.shape[1])
    scat2 = _make_scatter(npad, epad_s, oc_pad)

    degp = count(rowp)
    xw = _tc_matmul(xpad, W1)
    hp1 = _tc_scale(xw, degp)
    s1 = scat1(hp1, rowp, colp)
    hp2 = _tc_layer(s1, hp1, degp, b1r, W2p)
    s2 = scat2(hp2, rowp, colp)
    outp = _tc_finish(s2, hp2, degp, b2r)
    return outp[:n, :out_ch]


# speedup vs baseline: 4.0719x; 1.6817x over previous
import functools

import jax
import jax.numpy as jnp
from jax import lax
from jax.experimental import pallas as pl
from jax.experimental.pallas import tpu as pltpu
from jax.experimental.pallas import tpu_sc as plsc

NC = 2
NS = 16
NW = NC * NS
CHUNK = 128
NBUF = 3

_mesh = plsc.VectorSubcoreMesh(core_axis_name="c", subcore_axis_name="s")


def _round_up(a, m):
    return ((a + m - 1) // m) * m


def _fill(buf, rows, d, value):
    @pl.loop(0, rows)
    def _(i):
        @pl.loop(0, d // 16)
        def _(j):
            buf[pl.ds(i, 1), pl.ds(j * 16, 16)] = jnp.full((1, 16), value,
                                                           jnp.float32)


def _zero_stripe(zbuf, acc, sid, stripe, d, chunk=CHUNK):
    full, rem = stripe // chunk, stripe % chunk

    @pl.loop(0, full)
    def _(t):
        pltpu.sync_copy(zbuf, acc.at[pl.ds(sid * stripe + t * chunk, chunk)])

    if rem:
        pltpu.sync_copy(zbuf.at[pl.ds(0, rem)],
                        acc.at[pl.ds(sid * stripe + full * chunk, rem)])


def _read_stripe(acc, out_hbm, cid, sid, stripe):
    full, rem = stripe // CHUNK, stripe % CHUNK

    @pl.loop(0, full)
    def _(t):
        r0 = sid * stripe + t * CHUNK
        pltpu.sync_copy(acc.at[pl.ds(r0, CHUNK)], out_hbm.at[cid, pl.ds(r0, CHUNK)])

    if rem:
        r0 = sid * stripe + full * CHUNK
        pltpu.sync_copy(acc.at[pl.ds(r0, rem)], out_hbm.at[cid, pl.ds(r0, rem)])


def _make_count(npad, epad):
    per_w = epad // NW
    n_chunks = per_w // CHUNK
    n_rounds = n_chunks // NBUF
    stripe = npad // NS
    d = 128

    @functools.partial(
        pl.kernel,
        out_type=jax.ShapeDtypeStruct((NC, npad, d), jnp.float32),
        mesh=_mesh,
        scratch_types=[pltpu.VMEM((CHUNK, d), jnp.float32),
                       pltpu.VMEM_SHARED((npad, d), jnp.float32)]
        + [pltpu.VMEM((CHUNK,), jnp.int32) for _ in range(NBUF)]
        + [pltpu.SemaphoreType.DMA for _ in range(2 * NBUF)],
    )
    def count(row_hbm, out_hbm, ones_v, acc, *rest):
        ridx = rest[:NBUF]
        risem = rest[NBUF:2 * NBUF]
        ssem = rest[2 * NBUF:]

        cid = lax.axis_index("c")
        sid = lax.axis_index("s")
        wid = sid * NC + cid
        base = wid * per_w

        for b in range(NBUF):
            pltpu.async_copy(row_hbm.at[pl.ds(base + b * CHUNK, CHUNK)],
                             ridx[b], risem[b])

        _fill(ones_v, CHUNK, d, 0.0)
        _zero_stripe(ones_v, acc, sid, stripe, d)
        _fill(ones_v, CHUNK, d, 1.0 / d)
        plsc.subcore_barrier()

        @pl.loop(0, n_rounds)
        def _(r):
            c0 = r * NBUF
            for b in range(NBUF):
                c = c0 + b
                pltpu.make_async_copy(
                    row_hbm.at[pl.ds(base + c * CHUNK, CHUNK)], ridx[b],
                    risem[b]).wait()
                pltpu.async_copy(ones_v, acc.at[ridx[b]], ssem[b], add=True)
            for b in range(NBUF):
                c = c0 + b

                @pl.when(r < n_rounds - 1)
                def _():
                    pltpu.make_async_copy(ones_v, acc.at[ridx[b]],
                                          ssem[b]).wait()
                    pltpu.async_copy(
                        row_hbm.at[pl.ds(base + (c + NBUF) * CHUNK, CHUNK)],
                        ridx[b], risem[b])

        for b in range(NBUF):
            pltpu.make_async_copy(ones_v, acc.at[ridx[b]], ssem[b]).wait()

        plsc.subcore_barrier()
        _read_stripe(acc, out_hbm, cid, sid, stripe)

    return count


def _make_scatter(npad, epad, d, chunk=48, nbuf=7):
    per_w = epad // NW
    n_chunks = per_w // chunk
    n_rounds = n_chunks // nbuf
    stripe = npad // NS
    CHUNK, NBUF = chunk, nbuf

    @functools.partial(
        pl.kernel,
        out_type=jax.ShapeDtypeStruct((NC, npad, d), jnp.float32),
        mesh=_mesh,
        scratch_types=[pltpu.VMEM_SHARED((npad, d), jnp.float32)]
        + [pltpu.VMEM((CHUNK, d), jnp.float32) for _ in range(NBUF)]
        + [pltpu.VMEM((CHUNK,), jnp.int32) for _ in range(2 * NBUF)]
        + [pltpu.SemaphoreType.DMA for _ in range(4 * NBUF)],
    )
    def scatter(h_hbm, row_hbm, col_hbm, out_hbm, acc, *rest):
        rows = rest[:NBUF]
        ridx = rest[NBUF:2 * NBUF]
        cidx = rest[2 * NBUF:3 * NBUF]
        risem = rest[3 * NBUF:4 * NBUF]
        cisem = rest[4 * NBUF:5 * NBUF]
        gsem = rest[5 * NBUF:6 * NBUF]
        ssem = rest[6 * NBUF:]

        cid = lax.axis_index("c")
        sid = lax.axis_index("s")
        wid = sid * NC + cid
        base = wid * per_w

        def g_cp(b):
            return pltpu.make_async_copy(h_hbm.at[ridx[b]], rows[b], gsem[b])

        def s_cp(b):
            return pltpu.make_async_copy(rows[b], acc.at[cidx[b]], ssem[b])

        for b in range(NBUF):
            pltpu.async_copy(row_hbm.at[pl.ds(base + b * CHUNK, CHUNK)],
                             ridx[b], risem[b])
            pltpu.async_copy(col_hbm.at[pl.ds(base + b * CHUNK, CHUNK)],
                             cidx[b], cisem[b])

        _fill(rows[0], CHUNK, d, 0.0)
        _zero_stripe(rows[0], acc, sid, stripe, d, chunk=CHUNK)

        for b in range(NBUF):
            pltpu.make_async_copy(
                row_hbm.at[pl.ds(base + b * CHUNK, CHUNK)], ridx[b],
                risem[b]).wait()
            pltpu.async_copy(h_hbm.at[ridx[b]], rows[b], gsem[b])

        plsc.subcore_barrier()

        @pl.loop(0, n_rounds)
        def _(r):
            c0 = r * NBUF
            for b in range(NBUF):
                c = c0 + b
                g_cp(b).wait()
                pltpu.make_async_copy(
                    col_hbm.at[pl.ds(base + c * CHUNK, CHUNK)], cidx[b],
                    cisem[b]).wait()
                pltpu.async_copy(rows[b], acc.at[cidx[b]], ssem[b], add=True)

                @pl.when(r < n_rounds - 1)
                def _():
                    pltpu.async_copy(
                        row_hbm.at[pl.ds(base + (c + NBUF) * CHUNK, CHUNK)],
                        ridx[b], risem[b])

            for b in range(NBUF):
                c = c0 + b

                @pl.when(r < n_rounds - 1)
                def _():
                    s_cp(b).wait()
                    pltpu.async_copy(
                        col_hbm.at[pl.ds(base + (c + NBUF) * CHUNK, CHUNK)],
                        cidx[b], cisem[b])
                    pltpu.make_async_copy(
                        row_hbm.at[pl.ds(base + (c + NBUF) * CHUNK, CHUNK)],
                        ridx[b], risem[b]).wait()
                    pltpu.async_copy(h_hbm.at[ridx[b]], rows[b], gsem[b])

        for b in range(NBUF):
            s_cp(b).wait()

        plsc.subcore_barrier()
        _read_stripe(acc, out_hbm, cid, sid, stripe)

    return scatter


def _tc_matmul(x, w):
    def body(x_ref, w_ref, o_ref):
        o_ref[...] = jnp.dot(x_ref[...], w_ref[...],
                             preferred_element_type=jnp.float32)

    return pl.pallas_call(
        body,
        out_shape=jax.ShapeDtypeStruct((x.shape[0], w.shape[1]), jnp.float32),
    )(x, w)


def _dis(degp_ref):
    deg = jnp.sum(degp_ref[0] + degp_ref[1], axis=1, keepdims=True) + 1.0
    return lax.rsqrt(deg)


def _tc_scale(xw, degp):
    def body(xw_ref, degp_ref, o_ref):
        o_ref[...] = _dis(degp_ref) * xw_ref[...]

    return pl.pallas_call(
        body,
        out_shape=jax.ShapeDtypeStruct(xw.shape, jnp.float32),
    )(xw, degp)


def _tc_layer(sp, hp, degp, b, w):
    def body(sp_ref, hp_ref, degp_ref, b_ref, w_ref, o_ref):
        dis = _dis(degp_ref)
        aggr = dis * (sp_ref[0] + sp_ref[1] + hp_ref[...]) + b_ref[...]
        h = jnp.maximum(aggr, 0.0)
        o_ref[...] = dis * jnp.dot(h, w_ref[...],
                                   preferred_element_type=jnp.float32)

    return pl.pallas_call(
        body,
        out_shape=jax.ShapeDtypeStruct((hp.shape[0], w.shape[1]), jnp.float32),
    )(sp, hp, degp, b, w)


def _tc_finish(sp, hp, degp, b):
    def body(sp_ref, hp_ref, degp_ref, b_ref, o_ref):
        dis = _dis(degp_ref)
        o_ref[...] = dis * (sp_ref[0] + sp_ref[1] + hp_ref[...]) + b_ref[...]

    return pl.pallas_call(
        body,
        out_shape=jax.ShapeDtypeStruct(hp.shape, jnp.float32),
    )(sp, hp, degp, b)


def kernel(x, edge_index, W1, b1, W2, b2):
    n, in_ch = x.shape
    e = edge_index.shape[1]
    npad = _round_up(n + 1, CHUNK)
    epad_c = _round_up(e, NW * CHUNK * NBUF)
    epad_s = _round_up(e, NW * 48 * 7)
    epad = max(epad_c, epad_s)

    out_ch = W2.shape[1]
    oc_pad = _round_up(out_ch, 128)

    xpad = jnp.zeros((npad, in_ch), x.dtype).at[:n].set(x)
    pad_idx = n + (jnp.arange(epad - e, dtype=jnp.int32) % (npad - n))
    rowp = jnp.concatenate([edge_index[0], pad_idx])
    colp = jnp.concatenate([edge_index[1], pad_idx])
    b1r = b1.reshape(1, -1)
    b2r = jnp.zeros((1, oc_pad), b2.dtype).at[0, :out_ch].set(b2)
    W2p = jnp.zeros((W2.shape[0], oc_pad), W2.dtype).at[:, :out_ch].set(W2)

    count = _make_count(npad, epad_c)
    scat1 = _make_scatter(npad, epad_s, W1.shape[1])
    scat2 = _make_scatter(npad, epad_s, oc_pad)

    degp = count(rowp)
    xw = _tc_matmul(xpad, W1)
    hp1 = _tc_scale(xw, degp)
    s1 = scat1(hp1, rowp, colp)
    hp2 = _tc_layer(s1, hp1, degp, b1r, W2p)
    s2 = scat2(hp2, rowp, colp)
    outp = _tc_finish(s2, hp2, degp, b2r)
    return outp[:n, :out_ch]
